# baseline (speedup 1.0000x reference)
import functools

import jax
import jax.numpy as jnp
from jax import lax
from jax.experimental import pallas as pl
from jax.experimental.pallas import tpu as pltpu
from jax.experimental.pallas import tpu_sc as plsc

C = 192
H = W = 384
GH = GW = 384
N = GH * GW
NPIX = H * W
NC, NS = 2, 16
NW = NC * NS
PTS_PER_W = N // NW
CHUNK = 32
NCHUNK = PTS_PER_W // CHUNK
GL = 4 * CHUNK
SB = NCHUNK // 4
QB = 4 * CHUNK


def _sc_sample(table, idxc, wc):
    mesh = plsc.VectorSubcoreMesh(core_axis_name="c", subcore_axis_name="s")

    @functools.partial(
        pl.kernel,
        out_type=jax.ShapeDtypeStruct((C, N), jnp.float32),
        mesh=mesh,
        scratch_types=[
            pltpu.VMEM((SB, GL), jnp.int32),
            pltpu.VMEM((SB, GL), jnp.float32),
            pltpu.VMEM((GL, C), jnp.float32),
            pltpu.VMEM((GL, C), jnp.float32),
            pltpu.VMEM((C, QB), jnp.float32),
            pltpu.VMEM((C, QB), jnp.float32),
            pltpu.SemaphoreType.DMA,
            pltpu.SemaphoreType.DMA,
        ],
        compiler_params=pltpu.CompilerParams(use_tc_tiling_on_sc=False,
                                             needs_layout_passes=False),
    )
    def k(table_hbm, idx_hbm, w_hbm, out_hbm,
          stage_i, stage_w, rows_a, rows_b, out_a, out_b, sem_g, sem_o):
        wid = lax.axis_index("s") * NC + lax.axis_index("c")
        wbase = wid * PTS_PER_W
        rows_bufs = (rows_a, rows_b)
        out_bufs = (out_a, out_b)
        pt_idx = [[lax.iota(jnp.int32, 16) + (j * CHUNK + h * 16)
                   for h in range(2)] for j in range(4)]

        def refresh_i(ci):
            pltpu.sync_copy(idx_hbm.at[pl.ds(wid * NCHUNK + ci, SB)], stage_i)

        def refresh_w(ci):
            pltpu.sync_copy(w_hbm.at[pl.ds(wid * NCHUNK + ci, SB)], stage_w)

        def fire(ci, par):
            pltpu.async_copy(
                table_hbm.at[stage_i.at[lax.rem(ci, SB)]],
                rows_bufs[par], sem_g)

        def wait_gather(par):
            pltpu.make_async_copy(
                table_hbm.at[stage_i.at[0]], rows_bufs[par], sem_g).wait()

        def wait_write():
            pltpu.make_async_copy(
                out_a, out_hbm.at[:, pl.ds(0, QB)], sem_o).wait()

        def step(it, q):
            ci = 8 * it + q
            par = q % 2
            rows_v = rows_bufs[par]
            out_v = out_bufs[q // 4]
            slot = (q % 4) * CHUNK

            wait_gather(par)

            @pl.when(jnp.logical_and(lax.rem(ci + 1, SB) == 0,
                                     ci + 1 < NCHUNK))
            def _():
                refresh_i(ci + 1)

            @pl.when(ci + 1 < NCHUNK)
            def _():
                fire(ci + 1, 1 - par)

            if q % 4 == 0:
                @pl.when(it >= 1)
                def _():
                    wait_write()

            sw = lax.rem(ci, SB)
            wv = [[stage_w[sw, pl.ds(j * CHUNK + h * 16, 16)]
                   for h in range(2)] for j in range(4)]

            @plsc.parallel_loop(0, C, unroll=2)
            def ch_body(c):
                cvec = jnp.full((16,), c, jnp.int32)
                for h in range(2):
                    v = [plsc.load_gather(rows_v, [pt_idx[j][h], cvec])
                         for j in range(4)]
                    out_v[c, pl.ds(slot + h * 16, 16)] = (
                        v[0] * wv[0][h] + v[1] * wv[1][h]
                        + v[2] * wv[2][h] + v[3] * wv[3][h])

            if q % 4 == 3:
                qbase = wbase + (8 * it + (q - 3)) * CHUNK
                pltpu.async_copy(
                    out_v, out_hbm.at[:, pl.ds(qbase, QB)], sem_o)

        refresh_i(0)
        refresh_w(0)
        fire(0, 0)

        def body(it, carry):
            for q in range(8):
                ci = 8 * it + q

                @pl.when(jnp.logical_and(lax.rem(ci, SB) == 0, ci > 0))
                def _():
                    refresh_w(ci)

                step(it, q)
            return carry

        lax.fori_loop(0, NCHUNK // 8, body, 0)
        wait_write()
        wait_write()

    return k(table, idxc, wc)


def kernel(grid, matrix):
    x = grid[0, :, :, 0].reshape(-1)
    y = grid[0, :, :, 1].reshape(-1)
    ix = jnp.clip(((x + 1.0) * W - 1.0) / 2.0, 0.0, W - 1.0)
    iy = jnp.clip(((y + 1.0) * H - 1.0) / 2.0, 0.0, H - 1.0)
    ix0f = jnp.floor(ix)
    iy0f = jnp.floor(iy)
    wx = ix - ix0f
    wy = iy - iy0f
    ix0 = jnp.clip(ix0f.astype(jnp.int32), 0, W - 1)
    ix1 = jnp.clip(ix0 + 1, 0, W - 1)
    iy0 = jnp.clip(iy0f.astype(jnp.int32), 0, H - 1)
    iy1 = jnp.clip(iy0 + 1, 0, H - 1)
    idx4 = jnp.stack([iy0 * W + ix0, iy0 * W + ix1,
                      iy1 * W + ix0, iy1 * W + ix1])
    w4 = jnp.stack([(1.0 - wy) * (1.0 - wx), (1.0 - wy) * wx,
                    wy * (1.0 - wx), wy * wx])
    idxc = idx4.reshape(4, N // CHUNK, CHUNK).transpose(1, 0, 2).reshape(
        N // CHUNK, GL)
    wc = w4.reshape(4, N // CHUNK, CHUNK).transpose(1, 0, 2).reshape(
        N // CHUNK, GL)
    table = matrix.reshape(C, NPIX).T
    out_cm = _sc_sample(table, idxc, wc)
    return out_cm.reshape(1, C, GH, GW)


# speedup vs baseline: 2.0849x; 2.0849x over previous
import functools

import jax
import jax.numpy as jnp
from jax import lax
from jax.experimental import pallas as pl
from jax.experimental.pallas import tpu as pltpu
from jax.experimental.pallas import tpu_sc as plsc

C = 192
H = W = 384
GH = GW = 384
N = GH * GW
NPIX = H * W
NC, NS = 2, 16
NW = NC * NS
PTS_PER_W = N // NW
CHUNK = 32
NCHUNK = PTS_PER_W // CHUNK
GL = 4 * CHUNK
CG = C // 16


def _sc_sample(table, idxc, wc):
    mesh = plsc.VectorSubcoreMesh(core_axis_name="c", subcore_axis_name="s")

    @functools.partial(
        pl.kernel,
        out_type=jax.ShapeDtypeStruct((N, C), jnp.float32),
        mesh=mesh,
        scratch_types=[
            pltpu.VMEM((NCHUNK, GL), jnp.int32),
            pltpu.VMEM((NCHUNK, GL), jnp.float32),
            pltpu.VMEM((2, GL, C), jnp.bfloat16),
            pltpu.VMEM((2, CHUNK, C), jnp.float32),
            pltpu.SemaphoreType.DMA,
            pltpu.SemaphoreType.DMA,
        ],
        compiler_params=pltpu.CompilerParams(use_tc_tiling_on_sc=False,
                                             needs_layout_passes=False),
    )
    def k(table_hbm, idx_hbm, w_hbm, out_hbm,
          stage_i, stage_w, rows_v, out_v, sem_g, sem_o):
        wid = lax.axis_index("s") * NC + lax.axis_index("c")
        wbase = wid * PTS_PER_W
        pltpu.sync_copy(idx_hbm.at[pl.ds(wid * NCHUNK, NCHUNK)], stage_i)
        pltpu.sync_copy(w_hbm.at[pl.ds(wid * NCHUNK, NCHUNK)], stage_w)

        def fire(ci, par):
            pltpu.async_copy(
                table_hbm.at[stage_i.at[ci]], rows_v.at[par], sem_g)

        def wait_gather(par):
            pltpu.make_async_copy(
                table_hbm.at[stage_i.at[0]], rows_v.at[par], sem_g).wait()

        def wait_write():
            pltpu.make_async_copy(
                out_v.at[0], out_hbm.at[pl.ds(0, CHUNK)], sem_o).wait()

        def step(ci, par):
            @pl.when(ci + 1 < NCHUNK)
            def _():
                fire(ci + 1, 1 - par)

            wait_gather(par)

            @pl.when(ci >= 2)
            def _():
                wait_write()

            ci16 = jnp.full((16,), ci, jnp.int32)

            @plsc.parallel_loop(0, CHUNK, unroll=2)
            def pt_body(i):
                w = [plsc.load_gather(stage_w,
                                      [ci16, jnp.full((16,), j * CHUNK + i,
                                                      jnp.int32)])
                     for j in range(4)]
                wp = [plsc.pack(wj, wj, format=plsc.PackFormat.INTERLEAVED)
                      for wj in w]
                for g in range(C // 32):
                    s = pl.ds(g * 32, 32)
                    acc = (rows_v[par, 0 * CHUNK + i, s] * wp[0]
                           + rows_v[par, 1 * CHUNK + i, s] * wp[1]
                           + rows_v[par, 2 * CHUNK + i, s] * wp[2]
                           + rows_v[par, 3 * CHUNK + i, s] * wp[3])
                    lo, hi = plsc.unpack(acc,
                                         format=plsc.PackFormat.INTERLEAVED)
                    out_v[par, i, pl.ds(g * 32, 16)] = lo
                    out_v[par, i, pl.ds(g * 32 + 16, 16)] = hi

            pltpu.async_copy(
                out_v.at[par], out_hbm.at[pl.ds(wbase + ci * CHUNK, CHUNK)],
                sem_o)

        fire(0, 0)

        def pair_body(it, carry):
            step(2 * it, 0)
            step(2 * it + 1, 1)
            return carry

        lax.fori_loop(0, NCHUNK // 2, pair_body, 0)
        wait_write()
        wait_write()

    return k(table, idxc, wc)


def kernel(grid, matrix):
    x = grid[0, :, :, 0].reshape(-1)
    y = grid[0, :, :, 1].reshape(-1)
    ix = jnp.clip(((x + 1.0) * W - 1.0) / 2.0, 0.0, W - 1.0)
    iy = jnp.clip(((y + 1.0) * H - 1.0) / 2.0, 0.0, H - 1.0)
    ix0f = jnp.floor(ix)
    iy0f = jnp.floor(iy)
    wx = ix - ix0f
    wy = iy - iy0f
    ix0 = jnp.clip(ix0f.astype(jnp.int32), 0, W - 1)
    ix1 = jnp.clip(ix0 + 1, 0, W - 1)
    iy0 = jnp.clip(iy0f.astype(jnp.int32), 0, H - 1)
    iy1 = jnp.clip(iy0 + 1, 0, H - 1)
    idx4 = jnp.stack([iy0 * W + ix0, iy0 * W + ix1,
                      iy1 * W + ix0, iy1 * W + ix1])
    w4 = jnp.stack([(1.0 - wy) * (1.0 - wx), (1.0 - wy) * wx,
                    wy * (1.0 - wx), wy * wx])
    idxc = idx4.reshape(4, N // CHUNK, CHUNK).transpose(1, 0, 2).reshape(
        N // CHUNK, GL)
    wc = w4.reshape(4, N // CHUNK, CHUNK).transpose(1, 0, 2).reshape(
        N // CHUNK, GL)
    mt = matrix.astype(jnp.bfloat16).reshape(C, NPIX).T
    table = mt.reshape(NPIX, C // 32, 2, 16).swapaxes(2, 3).reshape(NPIX, C)
    out_flat = _sc_sample(table, idxc, wc)
    return out_flat.T.reshape(1, C, GH, GW)


# speedup vs baseline: 2.8712x; 1.3772x over previous
import functools

import jax
import jax.numpy as jnp
from jax import lax
from jax.experimental import pallas as pl
from jax.experimental.pallas import tpu as pltpu
from jax.experimental.pallas import tpu_sc as plsc

C = 192
H = W = 384
GH = GW = 384
N = GH * GW
NPIX = H * W
NC, NS = 2, 16
NW = NC * NS
PTS_PER_W = N // NW
CHUNK = 32
NCHUNK = PTS_PER_W // CHUNK
GL = 4 * CHUNK
CG = C // 16


def _sc_sample(table, idxc, wc):
    mesh = plsc.VectorSubcoreMesh(core_axis_name="c", subcore_axis_name="s")

    @functools.partial(
        pl.kernel,
        out_type=jax.ShapeDtypeStruct((N, C), jnp.float32),
        mesh=mesh,
        scratch_types=[
            pltpu.VMEM((NCHUNK, GL), jnp.int32),
            pltpu.VMEM((NCHUNK, GL), jnp.float32),
            pltpu.VMEM((2, GL, C), jnp.float32),
            pltpu.VMEM((2, CHUNK, C), jnp.float32),
            pltpu.SemaphoreType.DMA,
            pltpu.SemaphoreType.DMA,
        ],
        compiler_params=pltpu.CompilerParams(use_tc_tiling_on_sc=False,
                                             needs_layout_passes=False),
    )
    def k(table_hbm, idx_hbm, w_hbm, out_hbm,
          stage_i, stage_w, rows_v, out_v, sem_g, sem_o):
        wid = lax.axis_index("s") * NC + lax.axis_index("c")
        wbase = wid * PTS_PER_W
        pltpu.sync_copy(idx_hbm.at[pl.ds(wid * NCHUNK, NCHUNK)], stage_i)
        pltpu.sync_copy(w_hbm.at[pl.ds(wid * NCHUNK, NCHUNK)], stage_w)

        def fire(ci, par):
            pltpu.async_copy(
                table_hbm.at[stage_i.at[ci]], rows_v.at[par], sem_g)

        def wait_gather(par):
            pltpu.make_async_copy(
                table_hbm.at[stage_i.at[0]], rows_v.at[par], sem_g).wait()

        def wait_write():
            pltpu.make_async_copy(
                out_v.at[0], out_hbm.at[pl.ds(0, CHUNK)], sem_o).wait()

        def step(ci, par):
            @pl.when(ci + 1 < NCHUNK)
            def _():
                fire(ci + 1, 1 - par)

            wait_gather(par)

            @pl.when(ci >= 2)
            def _():
                wait_write()

            ci16 = jnp.full((16,), ci, jnp.int32)

            @plsc.parallel_loop(0, CHUNK, unroll=2)
            def pt_body(i):
                w = [plsc.load_gather(stage_w,
                                      [ci16, jnp.full((16,), j * CHUNK + i,
                                                      jnp.int32)])
                     for j in range(4)]
                for g in range(CG):
                    s = pl.ds(g * 16, 16)
                    out_v[par, i, s] = (
                        rows_v[par, 0 * CHUNK + i, s] * w[0]
                        + rows_v[par, 1 * CHUNK + i, s] * w[1]
                        + rows_v[par, 2 * CHUNK + i, s] * w[2]
                        + rows_v[par, 3 * CHUNK + i, s] * w[3])

            pltpu.async_copy(
                out_v.at[par], out_hbm.at[pl.ds(wbase + ci * CHUNK, CHUNK)],
                sem_o)

        fire(0, 0)

        def pair_body(it, carry):
            step(2 * it, 0)
            step(2 * it + 1, 1)
            return carry

        lax.fori_loop(0, NCHUNK // 2, pair_body, 0)
        wait_write()
        wait_write()

    return k(table, idxc, wc)


def kernel(grid, matrix):
    x = grid[0, :, :, 0].reshape(-1)
    y = grid[0, :, :, 1].reshape(-1)
    ix = jnp.clip(((x + 1.0) * W - 1.0) / 2.0, 0.0, W - 1.0)
    iy = jnp.clip(((y + 1.0) * H - 1.0) / 2.0, 0.0, H - 1.0)
    ix0f = jnp.floor(ix)
    iy0f = jnp.floor(iy)
    wx = ix - ix0f
    wy = iy - iy0f
    ix0 = jnp.clip(ix0f.astype(jnp.int32), 0, W - 1)
    ix1 = jnp.clip(ix0 + 1, 0, W - 1)
    iy0 = jnp.clip(iy0f.astype(jnp.int32), 0, H - 1)
    iy1 = jnp.clip(iy0 + 1, 0, H - 1)
    idx4 = jnp.stack([iy0 * W + ix0, iy0 * W + ix1,
                      iy1 * W + ix0, iy1 * W + ix1])
    w4 = jnp.stack([(1.0 - wy) * (1.0 - wx), (1.0 - wy) * wx,
                    wy * (1.0 - wx), wy * wx])
    idxc = idx4.reshape(4, N // CHUNK, CHUNK).transpose(1, 0, 2).reshape(
        N // CHUNK, GL)
    wc = w4.reshape(4, N // CHUNK, CHUNK).transpose(1, 0, 2).reshape(
        N // CHUNK, GL)
    table = matrix.reshape(C, NPIX).T
    out_flat = _sc_sample(table, idxc, wc)
    return out_flat.T.reshape(1, C, GH, GW)
